# Initial kernel scaffold; baseline (speedup 1.0000x reference)
import jax
import jax.numpy as jnp
from jax.experimental import pallas as pl


def kernel(mlvl_feats1, mlvl_feats2, mlvl_feats3, mlvl_feats4, reference_points, pc_range, img_shape, lidar2img):
    raise NotImplementedError("write your pallas kernel here")


# speedup vs baseline: 1.5064x; 1.5064x over previous
import functools

import jax
import jax.numpy as jnp
from jax import lax
from jax.experimental import pallas as pl
from jax.experimental.pallas import tpu as pltpu
from jax.experimental.pallas import tpu_sc as plsc

NCAM = 6
NCH = 256
NQ = 900
QP = 912
LANES = 16
NQV = QP // LANES
LEVELS = [(64, 176), (32, 88), (16, 44), (8, 22)]
HWS = [h * w for h, w in LEVELS]
FOFF = [0]
for _hw in HWS[:-1]:
    FOFF.append(FOFF[-1] + _hw)
FTOT = FOFF[-1] + HWS[-1]
EPS = 1e-5

NC, NS = 2, 16
NW = NC * NS
NITEMS = NCAM * NCH
IPW = NITEMS // NW


def _prelude_body(mm_ref, p8_ref, msk_ref, *iw_refs):
    idx_refs = iw_refs[:16]
    wgt_refs = iw_refs[16:]
    r = jnp.dot(mm_ref[...], p8_ref[...], preferred_element_type=jnp.float32,
                precision=jax.lax.Precision.HIGHEST)
    x = r[0:NCAM]
    y = r[8:8 + NCAM]
    z = r[16:16 + NCAM]
    denom = jnp.maximum(z, EPS)
    gx = (x / denom - 0.5) * 2.0
    gy = (y / denom - 0.5) * 2.0
    ok = ((z > EPS) & (gx > -1.0) & (gx < 1.0) & (gy > -1.0) & (gy < 1.0))
    msk_ref[...] = ok.astype(jnp.float32)
    qvalid = lax.broadcasted_iota(jnp.int32, (NCAM, QP), 1) < NQ
    for l, (hh, ww) in enumerate(LEVELS):
        xl = (gx + 1.0) * 0.5 * ww - 0.5
        yl = (gy + 1.0) * 0.5 * hh - 0.5
        xl = jnp.clip(xl, -2.0, ww + 1.0)
        yl = jnp.clip(yl, -2.0, hh + 1.0)
        x0 = jnp.floor(xl)
        y0 = jnp.floor(yl)
        wx1 = xl - x0
        wx0 = 1.0 - wx1
        wy1 = yl - y0
        wy0 = 1.0 - wy1
        corners = ((0.0, 0.0, wx0 * wy0), (1.0, 0.0, wx1 * wy0),
                   (0.0, 1.0, wx0 * wy1), (1.0, 1.0, wx1 * wy1))
        for k, (a, b, w) in enumerate(corners):
            cx = x0 + a
            cy = y0 + b
            valid = ((cx >= 0.0) & (cx <= ww - 1.0)
                     & (cy >= 0.0) & (cy <= hh - 1.0) & qvalid)
            ix = jnp.clip(cx, 0.0, ww - 1.0).astype(jnp.int32)
            iy = jnp.clip(cy, 0.0, hh - 1.0).astype(jnp.int32)
            idx_refs[l * 4 + k][...] = iy * ww + ix + FOFF[l]
            wgt_refs[l * 4 + k][...] = w * valid.astype(jnp.float32)


def _prelude(mm, p8):
    i32 = jnp.int32
    f32 = jnp.float32
    outs = ([jax.ShapeDtypeStruct((NCAM, QP), f32)]
            + [jax.ShapeDtypeStruct((NCAM, QP), i32) for _ in range(16)]
            + [jax.ShapeDtypeStruct((NCAM, QP), f32) for _ in range(16)])
    return pl.pallas_call(_prelude_body, out_shape=tuple(outs))(mm, p8)


def _sc_kernel(feats, idxs, wgts):
    mesh = plsc.VectorSubcoreMesh(core_axis_name="c", subcore_axis_name="s")

    @functools.partial(
        pl.kernel,
        out_type=jax.ShapeDtypeStruct((NCAM, NCH, 4, QP), jnp.float32),
        mesh=mesh,
        compiler_params=pltpu.CompilerParams(
            needs_layout_passes=False, use_tc_tiling_on_sc=False),
        scratch_types=[
            pltpu.VMEM((16, QP), jnp.int32),
            pltpu.VMEM((16, QP), jnp.float32),
            pltpu.VMEM((FTOT,), jnp.float32),
            pltpu.VMEM((4, QP), jnp.float32),
        ],
    )
    def body(*refs):
        f_refs = refs[0:4]
        i_refs = refs[4:20]
        w_refs = refs[20:36]
        out_hbm = refs[36]
        idx_v, wgt_v, feat_v, out_v = refs[37:41]
        wid = lax.axis_index("s") * NC + lax.axis_index("c")
        base = wid * IPW
        n0 = lax.shift_right_logical(base, 8)
        c0 = lax.bitwise_and(base, 255)

        def item_body(j, carry):
            n, c = carry

            @pl.when(jnp.logical_or(j == 0, c == 0))
            def _load_tables():
                for t in range(16):
                    pltpu.sync_copy(i_refs[t].at[n], idx_v.at[t])
                    pltpu.sync_copy(w_refs[t].at[n], wgt_v.at[t])

            for l in range(4):
                pltpu.sync_copy(f_refs[l].at[n, c],
                                feat_v.at[pl.ds(FOFF[l], HWS[l])])

            def qv_body(qv, _):
                s = qv * LANES
                for l in range(4):
                    acc = None
                    for k in range(4):
                        t = l * 4 + k
                        ii = idx_v[t, pl.ds(s, LANES)]
                        wk = wgt_v[t, pl.ds(s, LANES)]
                        vv = plsc.load_gather(feat_v, [ii])
                        acc = wk * vv if acc is None else acc + wk * vv
                    out_v[l, pl.ds(s, LANES)] = acc
                return 0

            lax.fori_loop(0, NQV, qv_body, 0)
            pltpu.sync_copy(out_v, out_hbm.at[n, c])
            cn = c + 1
            wrap = cn == NCH
            return (jnp.where(wrap, n + 1, n), jnp.where(wrap, 0, cn))

        lax.fori_loop(0, IPW, item_body, (n0, c0))

    return body(*feats, *idxs, *wgts)


def kernel(mlvl_feats1, mlvl_feats2, mlvl_feats3, mlvl_feats4,
           reference_points, pc_range, img_shape, lidar2img):
    f32 = jnp.float32
    a = lidar2img[0].astype(f32)
    sc = pc_range[3:6] - pc_range[0:3]
    off = pc_range[0:3]
    s_mat = jnp.concatenate(
        [jnp.concatenate([jnp.diag(sc), off[:, None]], axis=1),
         jnp.array([[0.0, 0.0, 0.0, 1.0]], dtype=f32)], axis=0)
    a2 = jnp.einsum("nij,jk->nik", a, s_mat)
    row_scale = jnp.stack([1.0 / img_shape[1], 1.0 / img_shape[0],
                           jnp.asarray(1.0, f32)]).astype(f32)
    a2 = a2[:, 0:3, :] * row_scale[None, :, None]
    mm = jnp.zeros((24, 8), f32)
    for i in range(3):
        mm = mm.at[8 * i:8 * i + NCAM, 0:4].set(a2[:, i, :])
    p8 = jnp.zeros((8, QP), f32)
    p8 = p8.at[0:3, :NQ].set(reference_points[0].T.astype(f32))
    p8 = p8.at[3, :NQ].set(1.0)

    pre = _prelude(mm, p8)
    msk = pre[0]
    idxs = pre[1:17]
    wgts = pre[17:33]

    feats = [mlvl_feats1.reshape(NCAM, NCH, HWS[0]),
             mlvl_feats2.reshape(NCAM, NCH, HWS[1]),
             mlvl_feats3.reshape(NCAM, NCH, HWS[2]),
             mlvl_feats4.reshape(NCAM, NCH, HWS[3])]
    out = _sc_kernel(feats, idxs, wgts)

    sampled = out[..., :NQ].transpose(1, 3, 0, 2).reshape(1, NCH, NQ, NCAM, 1, 4)
    mask = msk[:, :NQ].T.reshape(1, 1, NQ, NCAM, 1, 1)
    return reference_points, sampled, mask


# speedup vs baseline: 2.1911x; 1.4545x over previous
import functools

import jax
import jax.numpy as jnp
from jax import lax
from jax.experimental import pallas as pl
from jax.experimental.pallas import tpu as pltpu
from jax.experimental.pallas import tpu_sc as plsc

NCAM = 6
NCH = 256
NQ = 900
QP = 912
LANES = 16
NQV = QP // LANES
LEVELS = [(64, 176), (32, 88), (16, 44), (8, 22)]
HWS = [h * w for h, w in LEVELS]
FOFF = [0]
for _hw in HWS[:-1]:
    FOFF.append(FOFF[-1] + _hw)
FTOT = FOFF[-1] + HWS[-1]
EPS = 1e-5

NC, NS = 2, 16
NW = NC * NS
NITEMS = NCAM * NCH
IPW = NITEMS // NW


def _prelude_body(mm_ref, p8_ref, msk_ref, *iw_refs):
    idx_refs = iw_refs[:16]
    wgt_refs = iw_refs[16:]
    r = jnp.dot(mm_ref[...], p8_ref[...], preferred_element_type=jnp.float32,
                precision=jax.lax.Precision.HIGHEST)
    x = r[0:NCAM]
    y = r[8:8 + NCAM]
    z = r[16:16 + NCAM]
    denom = jnp.maximum(z, EPS)
    gx = (x / denom - 0.5) * 2.0
    gy = (y / denom - 0.5) * 2.0
    ok = ((z > EPS) & (gx > -1.0) & (gx < 1.0) & (gy > -1.0) & (gy < 1.0))
    msk_ref[...] = ok.astype(jnp.float32)
    qvalid = lax.broadcasted_iota(jnp.int32, (NCAM, QP), 1) < NQ
    for l, (hh, ww) in enumerate(LEVELS):
        xl = (gx + 1.0) * 0.5 * ww - 0.5
        yl = (gy + 1.0) * 0.5 * hh - 0.5
        xl = jnp.clip(xl, -2.0, ww + 1.0)
        yl = jnp.clip(yl, -2.0, hh + 1.0)
        x0 = jnp.floor(xl)
        y0 = jnp.floor(yl)
        wx1 = xl - x0
        wx0 = 1.0 - wx1
        wy1 = yl - y0
        wy0 = 1.0 - wy1
        corners = ((0.0, 0.0, wx0 * wy0), (1.0, 0.0, wx1 * wy0),
                   (0.0, 1.0, wx0 * wy1), (1.0, 1.0, wx1 * wy1))
        for k, (a, b, w) in enumerate(corners):
            cx = x0 + a
            cy = y0 + b
            valid = ((cx >= 0.0) & (cx <= ww - 1.0)
                     & (cy >= 0.0) & (cy <= hh - 1.0) & qvalid)
            ix = jnp.clip(cx, 0.0, ww - 1.0).astype(jnp.int32)
            iy = jnp.clip(cy, 0.0, hh - 1.0).astype(jnp.int32)
            idx_refs[l * 4 + k][...] = iy * ww + ix + FOFF[l]
            wgt_refs[l * 4 + k][...] = w * valid.astype(jnp.float32)


def _prelude(mm, p8):
    i32 = jnp.int32
    f32 = jnp.float32
    outs = ([jax.ShapeDtypeStruct((NCAM, QP), f32)]
            + [jax.ShapeDtypeStruct((NCAM, QP), i32) for _ in range(16)]
            + [jax.ShapeDtypeStruct((NCAM, QP), f32) for _ in range(16)])
    return pl.pallas_call(_prelude_body, out_shape=tuple(outs))(mm, p8)


NPAIR = NCH // 2
NP_ITEMS = NCAM * NPAIR
PPW = NP_ITEMS // NW
NJ2 = PPW // 2


def _sc_kernel(feats, idxs, wgts):
    mesh = plsc.VectorSubcoreMesh(core_axis_name="c", subcore_axis_name="s")

    @functools.partial(
        pl.kernel,
        out_type=jax.ShapeDtypeStruct((NCAM, NCH, 4, QP), jnp.float32),
        mesh=mesh,
        compiler_params=pltpu.CompilerParams(
            needs_layout_passes=False, use_tc_tiling_on_sc=False),
        scratch_types=[
            pltpu.VMEM((16, QP), jnp.int32),
            pltpu.VMEM((16, QP), jnp.float32),
            pltpu.VMEM((2 * FTOT,), jnp.float32),
            pltpu.VMEM((2 * FTOT,), jnp.float32),
            pltpu.VMEM((2, 4, QP), jnp.float32),
            pltpu.VMEM((2, 4, QP), jnp.float32),
            pltpu.SemaphoreType.DMA,
            pltpu.SemaphoreType.DMA,
            pltpu.SemaphoreType.DMA,
            pltpu.SemaphoreType.DMA,
        ],
    )
    def body(*refs):
        f_refs = refs[0:4]
        i_refs = refs[4:20]
        w_refs = refs[20:36]
        out_hbm = refs[36]
        idx_v, wgt_v, feat_a, feat_b, out_a, out_b = refs[37:43]
        sem_a, sem_b, sem_oa, sem_ob = refs[43:47]
        wid = lax.axis_index("s") * NC + lax.axis_index("c")
        base = wid * PPW
        n_init = lax.shift_right_logical(base, 7)
        p_init = lax.bitwise_and(base, NPAIR - 1)

        def inc(n, p):
            pn = p + 1
            wrap = pn == NPAIR
            n2 = jnp.where(wrap, n + 1, n)
            p2 = jnp.where(wrap, 0, pn)
            over = n2 == NCAM
            return jnp.where(over, NCAM - 1, n2), jnp.where(over, NPAIR - 1, p2)

        def issue_feats(n, p, fbuf, sem):
            c = p * 2
            for ch in range(2):
                for l in range(4):
                    pltpu.make_async_copy(
                        f_refs[l].at[n, c + ch],
                        fbuf.at[pl.ds(ch * FTOT + FOFF[l], HWS[l])],
                        sem).start()

        def drain_feats(fbuf, sem):
            for ch in range(2):
                for l in range(4):
                    pltpu.make_async_copy(
                        f_refs[l].at[0, 0],
                        fbuf.at[pl.ds(ch * FTOT + FOFF[l], HWS[l])],
                        sem).wait()

        def drain_out(obuf, sem):
            pltpu.make_async_copy(obuf, out_hbm.at[0, pl.ds(0, 2)], sem).wait()

        def load_tables(n):
            for t in range(16):
                pltpu.sync_copy(i_refs[t].at[n], idx_v.at[t])
                pltpu.sync_copy(w_refs[t].at[n], wgt_v.at[t])

        def compute(fbuf, obuf):
            def qv_body(qv, _):
                s = qv * LANES
                for l in range(4):
                    acc0 = None
                    acc1 = None
                    for k in range(4):
                        t = l * 4 + k
                        ii = idx_v[t, pl.ds(s, LANES)]
                        wk = wgt_v[t, pl.ds(s, LANES)]
                        v0 = plsc.load_gather(fbuf, [ii])
                        v1 = plsc.load_gather(fbuf, [ii + FTOT])
                        acc0 = wk * v0 if acc0 is None else acc0 + wk * v0
                        acc1 = wk * v1 if acc1 is None else acc1 + wk * v1
                    obuf[0, l, pl.ds(s, LANES)] = acc0
                    obuf[1, l, pl.ds(s, LANES)] = acc1
                return 0

            lax.fori_loop(0, NQV, qv_body, 0)

        issue_feats(n_init, p_init, feat_a, sem_a)

        def step(j2, carry):
            n0, p0 = carry
            n1, p1 = inc(n0, p0)
            n2, p2 = inc(n1, p1)
            issue_feats(n1, p1, feat_b, sem_b)

            @pl.when(jnp.logical_or(j2 == 0, p0 == 0))
            def _():
                load_tables(n0)

            drain_feats(feat_a, sem_a)

            @pl.when(j2 > 0)
            def _():
                drain_out(out_a, sem_oa)

            compute(feat_a, out_a)
            pltpu.make_async_copy(
                out_a, out_hbm.at[n0, pl.ds(p0 * 2, 2)], sem_oa).start()
            issue_feats(n2, p2, feat_a, sem_a)

            @pl.when(p1 == 0)
            def _():
                load_tables(n1)

            drain_feats(feat_b, sem_b)

            @pl.when(j2 > 0)
            def _():
                drain_out(out_b, sem_ob)

            compute(feat_b, out_b)
            pltpu.make_async_copy(
                out_b, out_hbm.at[n1, pl.ds(p1 * 2, 2)], sem_ob).start()
            return (n2, p2)

        lax.fori_loop(0, NJ2, step, (n_init, p_init))
        drain_feats(feat_a, sem_a)
        drain_out(out_a, sem_oa)
        drain_out(out_b, sem_ob)

    return body(*feats, *idxs, *wgts)


def kernel(mlvl_feats1, mlvl_feats2, mlvl_feats3, mlvl_feats4,
           reference_points, pc_range, img_shape, lidar2img):
    f32 = jnp.float32
    a = lidar2img[0].astype(f32)
    sc = pc_range[3:6] - pc_range[0:3]
    off = pc_range[0:3]
    s_mat = jnp.concatenate(
        [jnp.concatenate([jnp.diag(sc), off[:, None]], axis=1),
         jnp.array([[0.0, 0.0, 0.0, 1.0]], dtype=f32)], axis=0)
    a2 = jnp.einsum("nij,jk->nik", a, s_mat)
    row_scale = jnp.stack([1.0 / img_shape[1], 1.0 / img_shape[0],
                           jnp.asarray(1.0, f32)]).astype(f32)
    a2 = a2[:, 0:3, :] * row_scale[None, :, None]
    mm = jnp.zeros((24, 8), f32)
    for i in range(3):
        mm = mm.at[8 * i:8 * i + NCAM, 0:4].set(a2[:, i, :])
    p8 = jnp.zeros((8, QP), f32)
    p8 = p8.at[0:3, :NQ].set(reference_points[0].T.astype(f32))
    p8 = p8.at[3, :NQ].set(1.0)

    pre = _prelude(mm, p8)
    msk = pre[0]
    idxs = pre[1:17]
    wgts = pre[17:33]

    feats = [mlvl_feats1.reshape(NCAM, NCH, HWS[0]),
             mlvl_feats2.reshape(NCAM, NCH, HWS[1]),
             mlvl_feats3.reshape(NCAM, NCH, HWS[2]),
             mlvl_feats4.reshape(NCAM, NCH, HWS[3])]
    out = _sc_kernel(feats, idxs, wgts)

    sampled = out[..., :NQ].transpose(1, 3, 0, 2).reshape(1, NCH, NQ, NCAM, 1, 4)
    mask = msk[:, :NQ].T.reshape(1, 1, NQ, NCAM, 1, 1)
    return reference_points, sampled, mask


# speedup vs baseline: 2.2704x; 1.0362x over previous
import functools

import jax
import jax.numpy as jnp
from jax import lax
from jax.experimental import pallas as pl
from jax.experimental.pallas import tpu as pltpu
from jax.experimental.pallas import tpu_sc as plsc

NCAM = 6
NCH = 256
NQ = 900
QP = 912
LANES = 16
LEVELS = [(64, 176), (32, 88), (16, 44), (8, 22)]
HWS = [h * w for h, w in LEVELS]
EPS = 1e-5

NC, NS = 2, 16
NW = NC * NS
NITEMS = NCAM * NQ
OUT_ROWS = NITEMS * 8
IDX_PAD = 5440
NPAIR_STEPS = 85


def _prelude_body(mm_ref, p8_ref, pcr_ref, img_ref, msk_ref, *iw_refs):
    idx_refs = iw_refs[:16]
    wgt_refs = iw_refs[16:]
    xs = p8_ref[0:1, :] * (pcr_ref[3] - pcr_ref[0]) + pcr_ref[0]
    ys = p8_ref[1:2, :] * (pcr_ref[4] - pcr_ref[1]) + pcr_ref[1]
    zs = p8_ref[2:3, :] * (pcr_ref[5] - pcr_ref[2]) + pcr_ref[2]
    p4 = jnp.concatenate([xs, ys, zs, p8_ref[3:8, :]], axis=0)
    r = jnp.dot(mm_ref[...], p4, preferred_element_type=jnp.float32,
                precision=jax.lax.Precision.HIGHEST)
    x = r[0:NCAM]
    y = r[8:8 + NCAM]
    z = r[16:16 + NCAM]
    denom = jnp.maximum(z, EPS)
    gx = (x / denom / img_ref[1] - 0.5) * 2.0
    gy = (y / denom / img_ref[0] - 0.5) * 2.0
    ok = ((z > EPS) & (gx > -1.0) & (gx < 1.0) & (gy > -1.0) & (gy < 1.0))
    msk_ref[...] = ok.astype(jnp.float32)
    qvalid = lax.broadcasted_iota(jnp.int32, (NCAM, QP), 1) < NQ
    cam = lax.broadcasted_iota(jnp.int32, (NCAM, QP), 0)
    for l, (hh, ww) in enumerate(LEVELS):
        xl = (gx + 1.0) * 0.5 * ww - 0.5
        yl = (gy + 1.0) * 0.5 * hh - 0.5
        xl = jnp.clip(xl, -2.0, ww + 1.0)
        yl = jnp.clip(yl, -2.0, hh + 1.0)
        x0 = jnp.floor(xl)
        y0 = jnp.floor(yl)
        wx1 = xl - x0
        wx0 = 1.0 - wx1
        wy1 = yl - y0
        wy0 = 1.0 - wy1
        corners = ((0.0, 0.0, wx0 * wy0), (1.0, 0.0, wx1 * wy0),
                   (0.0, 1.0, wx0 * wy1), (1.0, 1.0, wx1 * wy1))
        for k, (a, b, w) in enumerate(corners):
            cx = x0 + a
            cy = y0 + b
            valid = ((cx >= 0.0) & (cx <= ww - 1.0)
                     & (cy >= 0.0) & (cy <= hh - 1.0) & qvalid)
            ix = jnp.clip(cx, 0.0, ww - 1.0).astype(jnp.int32)
            iy = jnp.clip(cy, 0.0, hh - 1.0).astype(jnp.int32)
            if l < 2:
                row = (((cam * hh + iy) * (ww // 8) + (ix // 8)) * 16
                       + (ix % 8))
            else:
                row = (((cam * ww + ix) * (hh // 8) + (iy // 8)) * 16
                       + (iy % 8))
            idx_refs[l * 4 + k][...] = row
            wgt_refs[l * 4 + k][...] = w * valid.astype(jnp.float32)


def _prelude(mm, p8, pcr, img):
    i32 = jnp.int32
    f32 = jnp.float32
    outs = ([jax.ShapeDtypeStruct((NCAM, QP), f32)]
            + [jax.ShapeDtypeStruct((NCAM, QP), i32) for _ in range(16)]
            + [jax.ShapeDtypeStruct((NCAM, QP), f32) for _ in range(16)])
    return pl.pallas_call(
        _prelude_body,
        out_shape=tuple(outs),
        in_specs=[pl.BlockSpec(),
                  pl.BlockSpec(),
                  pl.BlockSpec(memory_space=pltpu.SMEM),
                  pl.BlockSpec(memory_space=pltpu.SMEM)],
    )(mm, p8, pcr, img)


def _sc_kernel(tabs, idx_t, wgt_t):
    mesh = plsc.VectorSubcoreMesh(core_axis_name="c", subcore_axis_name="s")

    @functools.partial(
        pl.kernel,
        out_type=jax.ShapeDtypeStruct((OUT_ROWS, 128), jnp.float32),
        mesh=mesh,
        compiler_params=pltpu.CompilerParams(
            needs_layout_passes=False, use_tc_tiling_on_sc=False),
        scratch_types=[
            pltpu.VMEM((48, 128), jnp.int32),
            pltpu.VMEM((24, 128), jnp.float32),
            pltpu.VMEM((32, 128), jnp.float32),
            pltpu.VMEM((32, 128), jnp.float32),
            pltpu.VMEM((8, 128), jnp.float32),
            pltpu.VMEM((8, 128), jnp.float32),
            pltpu.SemaphoreType.DMA,
            pltpu.SemaphoreType.DMA,
            pltpu.SemaphoreType.DMA,
            pltpu.SemaphoreType.DMA,
        ],
    )
    def body(t1, t2, t3, t4, it, wt, out_hbm, idx_v, wgt_v,
             rows_a, rows_b, out_a, out_b, sga, sgb, soa, sob):
        t_refs = (t1, t2, t3, t4)
        wid = lax.axis_index("s") * NC + lax.axis_index("c")
        js = lax.shift_right_logical(wid * NITEMS, 5)
        je = lax.shift_right_logical((wid + 1) * NITEMS, 5)
        count = je - js
        r0i = lax.shift_right_logical(js, 2)
        r0w = lax.shift_right_logical(js, 3)
        pltpu.sync_copy(it.at[pl.ds(r0i, 48)], idx_v)
        pltpu.sync_copy(wt.at[pl.ds(r0w, 24)], wgt_v)
        ib0 = lax.bitwise_and(js, 3) * 32
        wb0 = lax.bitwise_and(js, 7) * 16

        def gather_descs(i, rows_buf, sem):
            tb = ib0 + i * 32
            row = lax.shift_right_logical(tb, 7)
            col = lax.bitwise_and(tb, 127)
            return [pltpu.make_async_copy(
                t_refs[l].at[idx_v.at[row, pl.ds(
                    pl.multiple_of(col + 8 * l, 8), 8)]],
                rows_buf.at[pl.ds(8 * l, 8)], sem) for l in range(4)]

        def compute(i, rows_buf, out_buf):
            tw = wb0 + i * 16
            roww = lax.shift_right_logical(tw, 7)
            colw = lax.bitwise_and(tw, 127)
            w16 = wgt_v[roww, pl.ds(colw, LANES)]
            for t in range(2):
                for l in range(4):
                    wks = [w16[l * 4 + k] for k in range(4)]
                    for v in range(8):
                        c = v * LANES
                        acc = None
                        for k in range(4):
                            rr = rows_buf[8 * l + 2 * k + t, pl.ds(c, LANES)]
                            term = wks[k] * rr
                            acc = term if acc is None else acc + term
                        out_buf[t * 4 + l, pl.ds(c, LANES)] = acc

        def out_desc(i, out_buf, sem):
            return pltpu.make_async_copy(
                out_buf, out_hbm.at[pl.ds((js + i) * 8, 8)], sem)

        def drain_out(out_buf, sem):
            pltpu.make_async_copy(out_buf, out_hbm.at[pl.ds(0, 8)], sem).wait()

        for d in gather_descs(0, rows_a, sga):
            d.start()

        def step(j2, carry):
            ia = 2 * j2
            ib = ia + 1
            ic = ia + 2

            @pl.when(ib < count)
            def _():
                for d in gather_descs(ib, rows_b, sgb):
                    d.start()

            @pl.when(ia < count)
            def _():
                for d in gather_descs(ia, rows_a, sga):
                    d.wait()

                @pl.when(j2 > 0)
                def _():
                    drain_out(out_a, soa)

                compute(ia, rows_a, out_a)
                out_desc(ia, out_a, soa).start()

            @pl.when(ic < count)
            def _():
                for d in gather_descs(ic, rows_a, sga):
                    d.start()

            @pl.when(ib < count)
            def _():
                for d in gather_descs(ib, rows_b, sgb):
                    d.wait()

                @pl.when(j2 > 0)
                def _():
                    drain_out(out_b, sob)

                compute(ib, rows_b, out_b)
                out_desc(ib, out_b, sob).start()

            return carry

        lax.fori_loop(0, NPAIR_STEPS, step, 0)
        drain_out(out_a, soa)
        drain_out(out_b, sob)

    return body(*tabs, idx_t, wgt_t)


def _feat_table(f, l):
    hh, ww = LEVELS[l]
    if l < 2:
        g = f.transpose(0, 1, 3, 4, 2)
        g = g.reshape(NCAM, hh, ww // 8, 8, 2, 128)
    else:
        g = f.transpose(0, 1, 4, 3, 2)
        g = g.reshape(NCAM, ww, hh // 8, 8, 2, 128)
    g = g.transpose(0, 1, 2, 4, 3, 5)
    return g.reshape(-1, 128)


def kernel(mlvl_feats1, mlvl_feats2, mlvl_feats3, mlvl_feats4,
           reference_points, pc_range, img_shape, lidar2img):
    f32 = jnp.float32
    a = lidar2img[0].astype(f32)
    mm = jnp.zeros((24, 8), f32)
    for i in range(3):
        mm = mm.at[8 * i:8 * i + NCAM, 0:4].set(a[:, i, :])
    p8 = jnp.zeros((8, QP), f32)
    p8 = p8.at[0:3, :NQ].set(reference_points[0].T.astype(f32))
    p8 = p8.at[3, :NQ].set(1.0)

    pre = _prelude(mm, p8, pc_range.astype(f32), img_shape.astype(f32))
    msk = pre[0]
    idxs = pre[1:17]
    wgts = pre[17:33]

    istk = jnp.stack(idxs)
    wstk = jnp.stack(wgts)
    iflat = istk.transpose(2, 1, 0)[:NQ].reshape(NITEMS, 16)
    wflat = wstk.transpose(2, 1, 0)[:NQ].reshape(NITEMS, 16)
    ichunk = jnp.stack([iflat, iflat + 8], axis=-1).reshape(NITEMS, 32)
    ichunk = jnp.pad(ichunk, ((0, IDX_PAD - NITEMS), (0, 0)))
    wflat = jnp.pad(wflat, ((0, IDX_PAD - NITEMS), (0, 0)))
    idx_t = ichunk.reshape(-1, 128)
    wgt_t = wflat.reshape(-1, 128)

    tabs = [_feat_table(f, l) for l, f in enumerate(
        [mlvl_feats1, mlvl_feats2, mlvl_feats3, mlvl_feats4])]

    out = _sc_kernel(tabs, idx_t, wgt_t)

    sampled = (out.reshape(NQ, NCAM, 2, 4, 128)
               .transpose(2, 4, 0, 1, 3)
               .reshape(1, NCH, NQ, NCAM, 1, 4))
    mask = msk[:, :NQ].T.reshape(1, 1, NQ, NCAM, 1, 1)
    return reference_points, sampled, mask
